# Initial kernel scaffold; baseline (speedup 1.0000x reference)
import jax
import jax.numpy as jnp
from jax.experimental import pallas as pl


def kernel(x, edge_index, W_l, b_l, W_r, W_out, b_out):
    raise NotImplementedError("write your pallas kernel here")


# speedup vs baseline: 4.0577x; 4.0577x over previous
import jax
import jax.numpy as jnp
from jax import lax
from jax.experimental import pallas as pl
from jax.experimental.pallas import tpu as pltpu
from jax.experimental.pallas import tpu_sc as plsc

_N = 10000
_E = 320000
_F = 128
_H = 16
_C = 40

_NW = 32
_EPT = 10240
_EPAD = _NW * _EPT
_KC = 1024
_NCHUNK = _EPT // _KC
_JROWS = _KC // 128
_NACC = 10240
_CROWS = _NACC // 128
_ZROWS = _NACC // 16


def _sc_body(x_hbm, src_hbm, dstf_hbm, dst2_hbm, zc_hbm,
             summed_hbm, counts_hbm,
             accum, srcv, dstf, dst2, rows, cnt, gsem, ssem):
    c = lax.axis_index("c")
    s = lax.axis_index("s")
    wid = s * 2 + c

    pltpu.sync_copy(zc_hbm, accum.at[pl.ds(s * _ZROWS, _ZROWS)])
    pltpu.sync_copy(zc_hbm.at[pl.ds(0, _CROWS)], cnt)
    plsc.subcore_barrier()

    ones = jnp.ones((16,), jnp.float32)
    sdescs = [None, None]
    for ch in range(_NCHUNK):
        off = wid * _EPT + ch * _KC
        row0 = wid * (_EPT // 128) + ch * _JROWS
        pltpu.sync_copy(src_hbm.at[pl.ds(off, _KC)], srcv)
        pltpu.sync_copy(dstf_hbm.at[pl.ds(off, _KC)], dstf)
        pltpu.sync_copy(dst2_hbm.at[pl.ds(row0, _JROWS)], dst2)
        for j in range(_JROWS):
            par = j % 2
            if sdescs[par] is not None:
                sdescs[par].wait()
            gd = pltpu.async_copy(x_hbm.at[srcv.at[pl.ds(j * 128, 128)]],
                                  rows.at[par], gsem)
            gd.wait()
            sdescs[par] = pltpu.async_copy(rows.at[par], accum.at[dst2.at[j]],
                                           ssem, add=True)

            @pl.loop(0, 8)
            def _count(g, j=j):
                idx = dstf[pl.ds(j * 128 + g * 16, 16)]
                plsc.addupdate_scatter(
                    cnt, [jnp.right_shift(idx, 7), jnp.bitwise_and(idx, 127)],
                    ones, mask=None)
    for d in sdescs:
        d.wait()

    plsc.subcore_barrier()

    @pl.when(s == 0)
    def _():
        pltpu.sync_copy(accum, summed_hbm.at[c])

    pltpu.sync_copy(cnt, counts_hbm.at[wid])


def _sc_aggregate(x, src, dstf, dst2, zc):
    mesh = plsc.VectorSubcoreMesh(core_axis_name="c", subcore_axis_name="s")
    return pl.kernel(
        _sc_body,
        out_type=(
            jax.ShapeDtypeStruct((2, _NACC, _F), jnp.float32),
            jax.ShapeDtypeStruct((_NW, _CROWS, 128), jnp.float32),
        ),
        mesh=mesh,
        scratch_types=[
            pltpu.VMEM_SHARED((_NACC, _F), jnp.float32),
            pltpu.VMEM((_KC,), jnp.int32),
            pltpu.VMEM((_KC,), jnp.int32),
            pltpu.VMEM((_JROWS, 128), jnp.int32),
            pltpu.VMEM((2, 128, _F), jnp.float32),
            pltpu.VMEM((_CROWS, 128), jnp.float32),
            pltpu.SemaphoreType.DMA,
            pltpu.SemaphoreType.DMA,
        ],
        compiler_params=pltpu.CompilerParams(needs_layout_passes=False),
    )(x, src, dstf, dst2, zc)


_BN = 2048


def _finish_body(s_ref, c_ref, x_ref, wl_ref, bl_ref, wr_ref, wo_ref, bo_ref,
                 o_ref):
    cnt = jnp.maximum(jnp.sum(c_ref[...], axis=0), 1.0)
    tot = (s_ref[0] + s_ref[1]).reshape(_BN // 128, 128, _F)
    mean = (tot / cnt[:, :, None]).reshape(_BN, _F)
    h = (lax.dot_general(mean, wl_ref[...], (((1,), (1,)), ((), ())),
                         preferred_element_type=jnp.float32)
         + lax.dot_general(x_ref[...], wr_ref[...], (((1,), (1,)), ((), ())),
                           preferred_element_type=jnp.float32)
         + bl_ref[...])
    h = jnp.maximum(h, 0.0)
    logits = lax.dot_general(h, wo_ref[...], (((1,), (1,)), ((), ())),
                             preferred_element_type=jnp.float32) + bo_ref[...]
    m = jnp.max(logits, axis=1, keepdims=True)
    ex = jnp.exp(logits - m)
    o_ref[...] = logits - m - jnp.log(jnp.sum(ex, axis=1, keepdims=True))


def _finish(summed, counts, x, W_l, b_l, W_r, W_out, b_out):
    grid = _NACC // _BN
    return pl.pallas_call(
        _finish_body,
        grid=(grid,),
        in_specs=[
            pl.BlockSpec((2, _BN, _F), lambda i: (0, i, 0)),
            pl.BlockSpec((_NW, _BN // 128, 128), lambda i: (0, i, 0)),
            pl.BlockSpec((_BN, _F), lambda i: (i, 0)),
            pl.BlockSpec((_H, _F), lambda i: (0, 0)),
            pl.BlockSpec((1, _H), lambda i: (0, 0)),
            pl.BlockSpec((_H, _F), lambda i: (0, 0)),
            pl.BlockSpec((_C, _H), lambda i: (0, 0)),
            pl.BlockSpec((1, _C), lambda i: (0, 0)),
        ],
        out_specs=pl.BlockSpec((_BN, _C), lambda i: (i, 0)),
        out_shape=jax.ShapeDtypeStruct((_N, _C), jnp.float32),
    )(summed, counts, x, W_l, b_l, W_r, W_out, b_out)


def kernel(x, edge_index, W_l, b_l, W_r, W_out, b_out):
    pad = _EPAD - _E
    src = jnp.concatenate([edge_index[0], jnp.zeros((pad,), jnp.int32)])
    dst = jnp.concatenate([edge_index[1], jnp.full((pad,), _N, jnp.int32)])
    dst2 = dst.reshape(_EPAD // 128, 128)
    zc = jnp.zeros((_ZROWS, _F), jnp.float32)

    summed, counts = _sc_aggregate(x, src, dst, dst2, zc)

    return _finish(summed, counts, x, W_l, b_l.reshape(1, _H), W_r,
                   W_out, b_out.reshape(1, _C))
